# baseline (speedup 1.0000x reference)
import functools

import jax
import jax.numpy as jnp
from jax import lax
from jax.experimental import pallas as pl
from jax.experimental.pallas import tpu as pltpu

_ROWS = 1000


def _mlp_body(scale_ref, x_ref, agg_ref, wa_ref, ba_ref, wb_ref, bb_ref,
              out_ref, *, final_relu):
    t = scale_ref[0] * x_ref[...] + agg_ref[...]
    h = jnp.dot(t, wa_ref[...], preferred_element_type=jnp.float32)
    h = jnp.maximum(h + ba_ref[...], 0.0)
    o = jnp.dot(h, wb_ref[...], preferred_element_type=jnp.float32)
    o = o + bb_ref[...]
    if final_relu:
        o = jnp.maximum(o, 0.0)
    out_ref[...] = o


def _gin_mlp(x, agg, scale, Wa, ba, Wb, bb, final_relu):
    n, din = x.shape
    dmid = Wa.shape[1]
    dout = Wb.shape[1]
    grid = (n // _ROWS,)
    return pl.pallas_call(
        functools.partial(_mlp_body, final_relu=final_relu),
        grid=grid,
        in_specs=[
            pl.BlockSpec(memory_space=pltpu.SMEM),
            pl.BlockSpec((_ROWS, din), lambda i: (i, 0)),
            pl.BlockSpec((_ROWS, din), lambda i: (i, 0)),
            pl.BlockSpec((din, dmid), lambda i: (0, 0)),
            pl.BlockSpec((1, dmid), lambda i: (0, 0)),
            pl.BlockSpec((dmid, dout), lambda i: (0, 0)),
            pl.BlockSpec((1, dout), lambda i: (0, 0)),
        ],
        out_specs=pl.BlockSpec((_ROWS, dout), lambda i: (i, 0)),
        out_shape=jax.ShapeDtypeStruct((n, dout), jnp.float32),
    )(scale, x, agg, Wa, ba.reshape(1, -1), Wb, bb.reshape(1, -1))


def _cluster_body(temp_ref, h_ref, mu_ref, out_ref):
    h = h_ref[...]
    nrm = jnp.sqrt(jnp.sum(h * h, axis=1, keepdims=True))
    data = h / (nrm + 1e-8)
    temp = temp_ref[0]

    def it(_, mu):
        dist = lax.dot_general(data, mu, (((1,), (1,)), ((), ())),
                               preferred_element_type=jnp.float32)
        logits = temp * dist
        m = jnp.max(logits, axis=1, keepdims=True)
        e = jnp.exp(logits - m)
        r = e / jnp.sum(e, axis=1, keepdims=True)
        cr = jnp.sum(r, axis=0)
        cm = lax.dot_general(r, data, (((0,), (0,)), ((), ())),
                             preferred_element_type=jnp.float32)
        return cm / (cr[:, None] + 1e-8)

    out_ref[...] = lax.fori_loop(0, 11, it, mu_ref[...])


def _cluster_mu(h1, mu_init, temp):
    k, nh = mu_init.shape
    n = h1.shape[0]
    return pl.pallas_call(
        _cluster_body,
        in_specs=[
            pl.BlockSpec(memory_space=pltpu.SMEM),
            pl.BlockSpec((n, nh), lambda: (0, 0)),
            pl.BlockSpec((k, nh), lambda: (0, 0)),
        ],
        out_specs=pl.BlockSpec((k, nh), lambda: (0, 0)),
        out_shape=jax.ShapeDtypeStruct((k, nh), jnp.float32),
    )(temp, h1, mu_init)


def _fin1_body(temp_ref, h1_ref, h2_ref, mu_ref, mskT_ref, sb1_ref, sb2_ref,
               s1_ref, s2_ref, csum_ref, msum_ref):
    h1 = h1_ref[...]
    h2 = h2_ref[...]
    mu = mu_ref[...]
    nrm = jnp.sqrt(jnp.sum(h1 * h1, axis=1, keepdims=True))
    data = h1 / (nrm + 1e-8)
    dist = lax.dot_general(data, mu, (((1,), (1,)), ((), ())),
                           preferred_element_type=jnp.float32)
    logits = temp_ref[0] * dist
    m = jnp.max(logits, axis=1, keepdims=True)
    e = jnp.exp(logits - m)
    s = e / jnp.sum(e, axis=1, keepdims=True)
    zt = jnp.dot(s, mu, preferred_element_type=jnp.float32)
    c2 = jax.nn.sigmoid(zt)
    s1_ref[...] = jnp.sum(h1 * c2, axis=1, keepdims=True) + sb1_ref[...]
    s2_ref[...] = jnp.sum(h2 * c2, axis=1, keepdims=True) + sb2_ref[...]

    mskT = mskT_ref[...]

    @pl.when(pl.program_id(0) == 0)
    def _():
        csum_ref[...] = jnp.zeros_like(csum_ref)
        msum_ref[...] = jnp.zeros_like(msum_ref)

    csum_ref[...] += jnp.sum(h1 * mskT, axis=0, keepdims=True)
    msum_ref[...] += jnp.sum(mskT, keepdims=True)


def _finals1(h1, h2, mu, mskT, sb1T, sb2T, temp):
    n, nh = h1.shape
    k = mu.shape[0]
    grid = (n // _ROWS,)
    return pl.pallas_call(
        _fin1_body,
        grid=grid,
        in_specs=[
            pl.BlockSpec(memory_space=pltpu.SMEM),
            pl.BlockSpec((_ROWS, nh), lambda i: (i, 0)),
            pl.BlockSpec((_ROWS, nh), lambda i: (i, 0)),
            pl.BlockSpec((k, nh), lambda i: (0, 0)),
            pl.BlockSpec((_ROWS, 1), lambda i: (i, 0)),
            pl.BlockSpec((_ROWS, 1), lambda i: (i, 0)),
            pl.BlockSpec((_ROWS, 1), lambda i: (i, 0)),
        ],
        out_specs=[
            pl.BlockSpec((_ROWS, 1), lambda i: (i, 0)),
            pl.BlockSpec((_ROWS, 1), lambda i: (i, 0)),
            pl.BlockSpec((1, nh), lambda i: (0, 0)),
            pl.BlockSpec((1, 1), lambda i: (0, 0)),
        ],
        out_shape=[
            jax.ShapeDtypeStruct((n, 1), jnp.float32),
            jax.ShapeDtypeStruct((n, 1), jnp.float32),
            jax.ShapeDtypeStruct((1, nh), jnp.float32),
            jax.ShapeDtypeStruct((1, 1), jnp.float32),
        ],
    )(temp, h1, h2, mu, mskT, sb1T, sb2T)


def _fin2_body(bd_ref, csum_ref, msum_ref, wd_ref, h1_ref, h2_ref, sb1_ref,
               sb2_ref, o1_ref, o2_ref):
    c = jax.nn.sigmoid(csum_ref[...] / msum_ref[...])
    q = lax.dot_general(wd_ref[...], c, (((1,), (1,)), ((), ())),
                        preferred_element_type=jnp.float32)
    bd = bd_ref[0]
    o1_ref[...] = jnp.dot(h1_ref[...], q,
                          preferred_element_type=jnp.float32) + bd + sb1_ref[...]
    o2_ref[...] = jnp.dot(h2_ref[...], q,
                          preferred_element_type=jnp.float32) + bd + sb2_ref[...]


def _finals2(h1, h2, csum, msum, Wd, bd, sb1T, sb2T):
    n, nh = h1.shape
    grid = (n // _ROWS,)
    return pl.pallas_call(
        _fin2_body,
        grid=grid,
        in_specs=[
            pl.BlockSpec(memory_space=pltpu.SMEM),
            pl.BlockSpec((1, nh), lambda i: (0, 0)),
            pl.BlockSpec((1, 1), lambda i: (0, 0)),
            pl.BlockSpec((nh, nh), lambda i: (0, 0)),
            pl.BlockSpec((_ROWS, nh), lambda i: (i, 0)),
            pl.BlockSpec((_ROWS, nh), lambda i: (i, 0)),
            pl.BlockSpec((_ROWS, 1), lambda i: (i, 0)),
            pl.BlockSpec((_ROWS, 1), lambda i: (i, 0)),
        ],
        out_specs=[
            pl.BlockSpec((_ROWS, 1), lambda i: (i, 0)),
            pl.BlockSpec((_ROWS, 1), lambda i: (i, 0)),
        ],
        out_shape=[
            jax.ShapeDtypeStruct((n, 1), jnp.float32),
            jax.ShapeDtypeStruct((n, 1), jnp.float32),
        ],
    )(bd, csum, msum, Wd, h1, h2, sb1T, sb2T)


def _edge_agg(x, src, dst):
    return jnp.zeros_like(x).at[dst].add(x[src])


def kernel(seq1, seq2, g, sparse, msk, samp_bias1, samp_bias2, cluster_temp,
           W1a, b1a, W1b, b1b, eps1, W2a, b2a, W2b, b2b, eps2, Wd, bd,
           mu_init):
    n = seq1.shape[1]
    src = g[0]
    dst = g[1]
    x1 = seq1[0]
    x2 = seq2[0]

    scale1 = (1.0 + eps1).reshape(1).astype(jnp.float32)
    scale2 = (1.0 + eps2).reshape(1).astype(jnp.float32)
    temp = jnp.asarray(cluster_temp, dtype=jnp.float32).reshape(1)
    bdv = jnp.asarray(bd, dtype=jnp.float32).reshape(1)

    xcat = jnp.concatenate([x1, x2], axis=0)
    agg0 = jnp.concatenate(
        [_edge_agg(x1, src, dst), _edge_agg(x2, src, dst)], axis=0)
    hcat = _gin_mlp(xcat, agg0, scale1, W1a, b1a, W1b, b1b, final_relu=True)

    agg1 = jnp.concatenate(
        [_edge_agg(hcat[:n], src, dst), _edge_agg(hcat[n:], src, dst)], axis=0)
    ocat = _gin_mlp(hcat, agg1, scale2, W2a, b2a, W2b, b2b, final_relu=False)
    h1 = ocat[:n]
    h2 = ocat[n:]

    mu = _cluster_mu(h1, mu_init, temp)

    mskT = msk.reshape(n, 1)
    sb1T = samp_bias1.reshape(n, 1)
    sb2T = samp_bias2.reshape(n, 1)
    s1, s2, csum, msum = _finals1(h1, h2, mu, mskT, sb1T, sb2T, temp)
    o1, o2 = _finals2(h1, h2, csum, msum, Wd, bdv, sb1T, sb2T)

    ret = jnp.concatenate([o1.reshape(1, n), o2.reshape(1, n)], axis=1)
    ret2 = jnp.concatenate([s1.reshape(1, n), s2.reshape(1, n)], axis=1)
    return (ret, ret2)
